# baseline (speedup 1.0000x reference)
import functools

import jax
import jax.numpy as jnp
from jax import lax
from jax.experimental import pallas as pl
from jax.experimental.pallas import tpu as pltpu
from jax.experimental.pallas import tpu_sc as plsc

_LAMDA = 0.9
_B = 16384
_C = 1000
_NC = 2
_NS = 16
_NW = _NC * _NS
_BPW = _B // _NW
_L = 16
_CHUNKS = _BPW // _L


def _loss_body(flat_hbm, action_hbm, delta_hbm, prop_hbm, out_hbm,
               act_v, delta_v, prop_v, idx_v, sel_v, part_v, sem):
    cid = lax.axis_index("c")
    sid = lax.axis_index("s")
    wid = sid * _NC + cid
    base = wid * _BPW

    pltpu.sync_copy(action_hbm.at[pl.ds(base, _BPW)], act_v)
    pltpu.sync_copy(delta_hbm.at[pl.ds(base, _BPW)], delta_v)
    pltpu.sync_copy(prop_hbm.at[pl.ds(base, _BPW)], prop_v)

    lane = lax.iota(jnp.int32, _L)
    for j in range(_CHUNKS):
        sl = pl.ds(j * _L, _L)
        rows = (base + j * _L) + lane
        idx_v[sl] = rows * _C + act_v[sl]

    pltpu.async_copy(flat_hbm.at[idx_v], sel_v, sem).wait()

    acc = jnp.zeros((_L,), jnp.float32)
    for j in range(_CHUNKS):
        sl = pl.ds(j * _L, _L)
        acc = acc + (-delta_v[sl] - _LAMDA) * (sel_v[sl] / prop_v[sl])
    part_v[...] = acc * (1.0 / _B)

    pltpu.sync_copy(part_v, out_hbm.at[wid])


@functools.partial(
    pl.kernel,
    out_type=jax.ShapeDtypeStruct((_NW, _L), jnp.float32),
    mesh=plsc.VectorSubcoreMesh(
        core_axis_name="c", subcore_axis_name="s",
        num_cores=_NC, num_subcores=_NS,
    ),
    scratch_types=[
        pltpu.VMEM((_BPW,), jnp.int32),
        pltpu.VMEM((_BPW,), jnp.float32),
        pltpu.VMEM((_BPW,), jnp.float32),
        pltpu.VMEM((_BPW,), jnp.int32),
        pltpu.VMEM((_BPW,), jnp.float32),
        pltpu.VMEM((_L,), jnp.float32),
        pltpu.SemaphoreType.DMA,
    ],
)
def _sc_loss(flat_hbm, action_hbm, delta_hbm, prop_hbm, out_hbm,
             act_v, delta_v, prop_v, idx_v, sel_v, part_v, sem):
    _loss_body(flat_hbm, action_hbm, delta_hbm, prop_hbm, out_hbm,
               act_v, delta_v, prop_v, idx_v, sel_v, part_v, sem)


@jax.jit
def kernel(output, action, delta, prop):
    flat = output.reshape(-1)
    act = action.astype(jnp.int32)
    parts = _sc_loss(flat, act, delta, prop)
    return jnp.sum(parts)


# speedup vs baseline: 1.3529x; 1.3529x over previous
import functools

import jax
import jax.numpy as jnp
from jax import lax
from jax.experimental import pallas as pl
from jax.experimental.pallas import tpu as pltpu
from jax.experimental.pallas import tpu_sc as plsc

_LAMDA = 0.9
_B = 16384
_C = 1000
_NC = 2
_NS = 16
_NW = _NC * _NS
_BPW = _B // _NW
_L = 16
_R = 32
_NBLK = _BPW // _R


def _loss_body(table_hbm, action_hbm, delta_hbm, prop_hbm, out_hbm,
               act_v, delta_v, prop_v, buf0, buf1, part_v, sem0, sem1):
    cid = lax.axis_index("c")
    sid = lax.axis_index("s")
    wid = sid * _NC + cid
    base = pl.multiple_of(wid * _BPW, _BPW)

    pltpu.sync_copy(action_hbm.at[pl.ds(base, _BPW)], act_v)
    pltpu.sync_copy(delta_hbm.at[pl.ds(base, _BPW)], delta_v)
    pltpu.sync_copy(prop_hbm.at[pl.ds(base, _BPW)], prop_v)

    bufs = [buf0, buf1]
    sems = [sem0, sem1]
    lane = lax.iota(jnp.int32, _L)

    def _start(b):
        return pltpu.async_copy(
            table_hbm.at[pl.ds(base + b * _R, _R), :], bufs[b % 2],
            sems[b % 2])

    copies = [None, None]
    copies[0] = _start(0)
    acc = jnp.zeros((_L,), jnp.float32)
    for b in range(_NBLK):
        copies[b % 2].wait()
        if b + 1 < _NBLK:
            copies[(b + 1) % 2] = _start(b + 1)
        buf = bufs[b % 2]
        for j in range(_R // _L):
            sl = pl.ds(b * _R + j * _L, _L)
            a = act_v[sl]
            loc = j * _L + lane
            sel = plsc.load_gather(buf, [loc, a])
            acc = acc + (-delta_v[sl] - _LAMDA) * (sel / prop_v[sl])
    part_v[...] = acc * (1.0 / _B)

    pltpu.sync_copy(part_v, out_hbm.at[wid])


@functools.partial(
    pl.kernel,
    out_type=jax.ShapeDtypeStruct((_NW, _L), jnp.float32),
    mesh=plsc.VectorSubcoreMesh(
        core_axis_name="c", subcore_axis_name="s",
        num_cores=_NC, num_subcores=_NS,
    ),
    scratch_types=[
        pltpu.VMEM((_BPW,), jnp.int32),
        pltpu.VMEM((_BPW,), jnp.float32),
        pltpu.VMEM((_BPW,), jnp.float32),
        pltpu.VMEM((_R, _C), jnp.float32),
        pltpu.VMEM((_R, _C), jnp.float32),
        pltpu.VMEM((_L,), jnp.float32),
        pltpu.SemaphoreType.DMA,
        pltpu.SemaphoreType.DMA,
    ],
    compiler_params=pltpu.CompilerParams(needs_layout_passes=False),
)
def _sc_loss(table_hbm, action_hbm, delta_hbm, prop_hbm, out_hbm,
             act_v, delta_v, prop_v, buf0, buf1, part_v, sem0, sem1):
    _loss_body(table_hbm, action_hbm, delta_hbm, prop_hbm, out_hbm,
               act_v, delta_v, prop_v, buf0, buf1, part_v, sem0, sem1)


@jax.jit
def kernel(output, action, delta, prop):
    act = action.astype(jnp.int32)
    parts = _sc_loss(output, act, delta, prop)
    return jnp.sum(parts)


# speedup vs baseline: 5.7258x; 4.2322x over previous
import functools

import jax
import jax.experimental.layout
import jax.numpy as jnp
from jax import lax
from jax.experimental import pallas as pl
from jax.experimental.pallas import tpu as pltpu
from jax.experimental.pallas import tpu_sc as plsc

_LAMDA = 0.9
_B = 16384
_C = 1000
_NC = 2
_NS = 16
_NW = _NC * _NS
_BPW = _B // _NW
_L = 16
_CHUNKS = _BPW // _L
_NLINES = (_C // 8) * (_B // 128) * 8


def _loss_body(lines_hbm, action_hbm, delta_hbm, prop_hbm, out_hbm,
               act_v, delta_v, prop_v, idx_v, lines_v, part_v, sem):
    cid = lax.axis_index("c")
    sid = lax.axis_index("s")
    wid = sid * _NC + cid
    base = pl.multiple_of(wid * _BPW, _BPW)

    pltpu.sync_copy(action_hbm.at[pl.ds(base, _BPW)], act_v)
    pltpu.sync_copy(delta_hbm.at[pl.ds(base, _BPW)], delta_v)
    pltpu.sync_copy(prop_hbm.at[pl.ds(base, _BPW)], prop_v)

    lane = lax.iota(jnp.int32, _L)
    for j in range(_CHUNKS):
        sl = pl.ds(j * _L, _L)
        a = act_v[sl]
        r = (base + j * _L) + lane
        ln = (lax.shift_right_logical(a, 3) * 1024
              + lax.shift_right_logical(r, 7) * 8
              + jnp.bitwise_and(a, 7))
        idx_v[sl] = ln

    pltpu.async_copy(lines_hbm.at[idx_v], lines_v, sem).wait()

    acc = jnp.zeros((_L,), jnp.float32)
    for j in range(_CHUNKS):
        sl = pl.ds(j * _L, _L)
        loc = j * _L + lane
        off = ((j * _L) % 128) + lane
        sel = plsc.load_gather(lines_v, [loc, off])
        acc = acc + (-delta_v[sl] - _LAMDA) * (sel / prop_v[sl])
    part_v[...] = acc * (1.0 / _B)

    pltpu.sync_copy(part_v, out_hbm.at[wid])


@functools.partial(
    pl.kernel,
    out_type=jax.ShapeDtypeStruct((_NW, _L), jnp.float32),
    mesh=plsc.VectorSubcoreMesh(
        core_axis_name="c", subcore_axis_name="s",
        num_cores=_NC, num_subcores=_NS,
    ),
    scratch_types=[
        pltpu.VMEM((_BPW,), jnp.int32),
        pltpu.VMEM((_BPW,), jnp.float32),
        pltpu.VMEM((_BPW,), jnp.float32),
        pltpu.VMEM((_BPW,), jnp.int32),
        pltpu.VMEM((_BPW, 128), jnp.float32),
        pltpu.VMEM((_L,), jnp.float32),
        pltpu.SemaphoreType.DMA,
    ],
    compiler_params=pltpu.CompilerParams(needs_layout_passes=False),
)
def _sc_loss(lines_hbm, action_hbm, delta_hbm, prop_hbm, out_hbm,
             act_v, delta_v, prop_v, idx_v, lines_v, part_v, sem):
    _loss_body(lines_hbm, action_hbm, delta_hbm, prop_hbm, out_hbm,
               act_v, delta_v, prop_v, idx_v, lines_v, part_v, sem)


@jax.jit
def kernel(output, action, delta, prop):
    t = output.T.reshape(_C // 8, 8, _B // 128, 128)
    lines = t.transpose(0, 2, 1, 3).reshape(_NLINES, 128)
    act = action.astype(jnp.int32)
    parts = _sc_loss(lines, act, delta, prop)
    return jnp.sum(parts)


# speedup vs baseline: 6.5548x; 1.1448x over previous
import functools

import jax
import jax.experimental.layout
import jax.numpy as jnp
from jax import lax
from jax.experimental import pallas as pl
from jax.experimental.pallas import tpu as pltpu
from jax.experimental.pallas import tpu_sc as plsc

_LAMDA = 0.9
_B = 16384
_C = 1000
_NC = 2
_NS = 16
_NW = _NC * _NS
_BPW = _B // _NW
_L = 16
_CHUNKS = _BPW // _L


def _loss_body(flat_hbm, action_hbm, delta_hbm, prop_hbm, out_hbm,
               act_v, delta_v, prop_v, idx_v, sel_v, part_v, sem, sem_in):
    cid = lax.axis_index("c")
    sid = lax.axis_index("s")
    wid = sid * _NC + cid
    base = pl.multiple_of(wid * _BPW, _BPW)

    dp_copies = [
        pltpu.async_copy(delta_hbm.at[pl.ds(base, _BPW)], delta_v, sem_in),
        pltpu.async_copy(prop_hbm.at[pl.ds(base, _BPW)], prop_v, sem_in),
    ]
    pltpu.sync_copy(action_hbm.at[pl.ds(base, _BPW)], act_v)

    lane = lax.iota(jnp.int32, _L)

    def _idx_body(j, carry):
        sl = pl.ds(j * _L, _L)
        a = act_v[sl]
        r = (base + j * _L) + lane
        wa = ((lax.shift_right_logical(a, 3) * 1024
               + lax.shift_right_logical(r, 7) * 8
               + jnp.bitwise_and(a, 7)) * 128
              + jnp.bitwise_and(r, 127))
        idx_v[sl] = wa
        return carry

    lax.fori_loop(0, _CHUNKS, _idx_body, 0, unroll=4)

    gather = pltpu.async_copy(flat_hbm.at[idx_v], sel_v, sem)
    for c in dp_copies:
        c.wait()
    gather.wait()

    def _acc_body(j, acc):
        sl = pl.ds(j * _L, _L)
        return acc + (-delta_v[sl] - _LAMDA) * (sel_v[sl] / prop_v[sl])

    acc = lax.fori_loop(0, _CHUNKS, _acc_body,
                        jnp.zeros((_L,), jnp.float32), unroll=4)
    part_v[...] = acc * (1.0 / _B)

    pltpu.sync_copy(part_v, out_hbm.at[wid])


@functools.partial(
    pl.kernel,
    out_type=jax.ShapeDtypeStruct((_NW, _L), jnp.float32),
    mesh=plsc.VectorSubcoreMesh(
        core_axis_name="c", subcore_axis_name="s",
        num_cores=_NC, num_subcores=_NS,
    ),
    scratch_types=[
        pltpu.VMEM((_BPW,), jnp.int32),
        pltpu.VMEM((_BPW,), jnp.float32),
        pltpu.VMEM((_BPW,), jnp.float32),
        pltpu.VMEM((_BPW,), jnp.int32),
        pltpu.VMEM((_BPW,), jnp.float32),
        pltpu.VMEM((_L,), jnp.float32),
        pltpu.SemaphoreType.DMA,
        pltpu.SemaphoreType.DMA,
    ],
    compiler_params=pltpu.CompilerParams(needs_layout_passes=False),
)
def _sc_loss(flat_hbm, action_hbm, delta_hbm, prop_hbm, out_hbm,
             act_v, delta_v, prop_v, idx_v, sel_v, part_v, sem, sem_in):
    _loss_body(flat_hbm, action_hbm, delta_hbm, prop_hbm, out_hbm,
               act_v, delta_v, prop_v, idx_v, sel_v, part_v, sem, sem_in)


@jax.jit
def kernel(output, action, delta, prop):
    t = output.T.reshape(_C // 8, 8, _B // 128, 128)
    flat = t.transpose(0, 2, 1, 3).reshape(_B * _C)
    act = action.astype(jnp.int32)
    parts = _sc_loss(flat, act, delta, prop)
    return jnp.sum(parts)
